# baseline (speedup 1.0000x reference)
import functools

import jax
import jax.numpy as jnp
from jax import lax
from jax.experimental import pallas as pl
from jax.experimental.pallas import tpu as pltpu
from jax.experimental.pallas import tpu_sc as plsc

_NC = 2
_NS = 16
_NW = _NC * _NS

_DEG = 16
_H = 32
_DF = 32
_PK = 4
_LW = _PK * _H
_WW = _LW // 2
_TWW = _H // 2


def _pack_pair(lo_f32, hi_f32):
    lo = lax.convert_element_type(
        lax.bitcast_convert_type(lo_f32.astype(jnp.bfloat16), jnp.int16), jnp.int32
    ) & jnp.int32(0xFFFF)
    hi = lax.convert_element_type(
        lax.bitcast_convert_type(hi_f32.astype(jnp.bfloat16), jnp.int16), jnp.int32
    ) << 16
    return lo | hi


def _unpack_pair(w_i32):
    lo = lax.bitcast_convert_type(w_i32 << 16, jnp.float32)
    hi = lax.bitcast_convert_type(w_i32 & jnp.int32(-65536), jnp.float32)
    return lo, hi


def _phase0_body(y_ref, fy_ref, w1a_ref, w1bs_ref, b1s_ref, t_ref, b_ref):
    yb = y_ref[...]
    a = jnp.dot(yb, w1a_ref[...], preferred_element_type=jnp.float32)
    fyb = fy_ref[...]
    t_ref[...] = jnp.concatenate(
        [
            _pack_pair(a[:, :_TWW], a[:, _TWW:]),
            _pack_pair(fyb[:, :_TWW], fyb[:, _TWW:]),
        ],
        axis=1,
    )
    b_ref[...] = (
        jnp.dot(yb, w1bs_ref[...], preferred_element_type=jnp.float32) + b1s_ref[...]
    ).astype(jnp.bfloat16)


def _phase2_body(gw_ref, b4_ref, w2s_ref, b2s_ref, out_ref):
    nb = b4_ref.shape[0]
    b4 = b4_ref[...].astype(jnp.float32)
    w2s = w2s_ref[...]
    b2s = b2s_ref[...]
    acc = jnp.zeros((nb, _LW), jnp.float32)
    for c in range(_DEG // _PK):
        lo, hi = _unpack_pair(gw_ref[pl.ds(c * nb, nb), :])
        h = jax.nn.gelu(
            jnp.concatenate([lo[:, :_WW], hi[:, :_WW]], axis=1) + b4
        )
        k = jnp.dot(h, w2s, preferred_element_type=jnp.float32) + b2s
        acc = acc + k * jnp.concatenate([lo[:, _WW:], hi[:, _WW:]], axis=1)
    s_lo = (
        acc[:, 0:16] + acc[:, 16:32] + acc[:, 32:48] + acc[:, 48:64]
    )
    s_hi = (
        acc[:, 64:80] + acc[:, 80:96] + acc[:, 96:112] + acc[:, 112:128]
    )
    out_ref[...] = jnp.concatenate([s_lo, s_hi], axis=1) * (1.0 / _DEG)


def _sc_gather2(tt, idx_t, n_nodes, nb):
    n_edges = n_nodes * _DEG
    prows = n_edges // _PK
    prows_per_tile = prows // _NW
    prows_per_step = 256
    steps = prows_per_tile // prows_per_step
    cg = _DEG // _PK

    mesh = plsc.VectorSubcoreMesh(core_axis_name="c", subcore_axis_name="s")

    @functools.partial(
        pl.kernel,
        out_type=jax.ShapeDtypeStruct((prows, 2 * _WW), jnp.int32),
        mesh=mesh,
        scratch_types=[
            [pltpu.VMEM((prows_per_step,), jnp.int32) for _ in range(_PK)],
            pltpu.VMEM((_PK * prows_per_step, 2 * _TWW), jnp.int32),
            pltpu.SemaphoreType.DMA,
        ],
        compiler_params=pltpu.CompilerParams(use_tc_tiling_on_sc=False),
    )
    def gather_kernel(tt_hbm, idx_hbm, gw_hbm, idx_vs, rt_v, sem):
        wid = lax.axis_index("s") * _NC + lax.axis_index("c")
        prow_base = wid * prows_per_tile

        def step(t, carry):
            poff = prow_base + t * prows_per_step
            seg = poff // nb
            m0 = poff % nb
            c = seg % cg
            node0 = (seg // cg) * nb + m0
            for q in range(_PK):
                pltpu.sync_copy(
                    idx_hbm.at[c * _PK + q, pl.ds(node0, prows_per_step)],
                    idx_vs[q],
                )
            cps = []
            for q in range(_PK):
                sl = pl.ds(q * prows_per_step, prows_per_step)
                cps.append(pltpu.async_copy(tt_hbm.at[idx_vs[q]], rt_v.at[sl], sem))
            for cp in cps:
                cp.wait()
            rows = pl.ds(poff, prows_per_step)
            for q in range(_PK):
                sl = pl.ds(q * prows_per_step, prows_per_step)
                pltpu.sync_copy(
                    rt_v.at[sl, pl.ds(0, _TWW)],
                    gw_hbm.at[rows, pl.ds(q * _TWW, _TWW)],
                )
                pltpu.sync_copy(
                    rt_v.at[sl, pl.ds(_TWW, _TWW)],
                    gw_hbm.at[rows, pl.ds(_WW + q * _TWW, _TWW)],
                )
            return carry

        lax.fori_loop(0, steps, step, None)

    return gather_kernel(tt, idx_t)


def kernel(y, f_y, neighbors_index, neighbors_row_splits, W1, b1, W2, b2):
    del neighbors_row_splits
    n = y.shape[0]
    idx = neighbors_index.astype(jnp.int32)
    nb = 1024
    n_pad = ((n + nb - 1) // nb) * nb
    y_p = jnp.pad(y, ((0, n_pad - n), (0, 0)))
    fy_p = jnp.pad(f_y, ((0, n_pad - n), (0, 0)))
    idx_t = jnp.pad(idx.reshape(n, _DEG), ((0, n_pad - n), (0, 0))).T

    featmap = jnp.array(
        [(l % 16) + (16 if l >= _WW else 0) for l in range(_LW)], dtype=jnp.int32
    )
    w1a = W1[:3]
    w1bs = W1[3:][:, featmap]
    b1s = b1[featmap].reshape(1, _LW)
    qmap = jnp.array(
        [(l // 16) % _PK for l in range(_LW)], dtype=jnp.int32
    )
    w2d = jnp.where(
        qmap[:, None] == qmap[None, :],
        W2[featmap[:, None], featmap[None, :]],
        0.0,
    ).astype(jnp.float32)
    b2s = b2[featmap].reshape(1, _LW)

    t_tab, b_tab = pl.pallas_call(
        _phase0_body,
        grid=(n_pad // nb,),
        in_specs=[
            pl.BlockSpec((nb, 3), lambda i: (i, 0)),
            pl.BlockSpec((nb, _DF), lambda i: (i, 0)),
            pl.BlockSpec((3, _H), lambda i: (0, 0)),
            pl.BlockSpec((3, _LW), lambda i: (0, 0)),
            pl.BlockSpec((1, _LW), lambda i: (0, 0)),
        ],
        out_specs=[
            pl.BlockSpec((nb, 2 * _TWW), lambda i: (i, 0)),
            pl.BlockSpec((nb, _LW), lambda i: (i, 0)),
        ],
        out_shape=[
            jax.ShapeDtypeStruct((n_pad, 2 * _TWW), jnp.int32),
            jax.ShapeDtypeStruct((n_pad, _LW), jnp.bfloat16),
        ],
    )(y_p, fy_p, w1a, w1bs, b1s)

    gw4 = _sc_gather2(t_tab, idx_t, n_pad, nb)

    nr = nb * _DEG // _PK
    out = pl.pallas_call(
        _phase2_body,
        grid=(n_pad // nb,),
        in_specs=[
            pl.BlockSpec((nr, 2 * _WW), lambda i: (i, 0)),
            pl.BlockSpec((nb, _LW), lambda i: (i, 0)),
            pl.BlockSpec((_LW, _LW), lambda i: (0, 0)),
            pl.BlockSpec((1, _LW), lambda i: (0, 0)),
        ],
        out_specs=pl.BlockSpec((nb, _DF), lambda i: (i, 0)),
        out_shape=jax.ShapeDtypeStruct((n_pad, _DF), jnp.float32),
    )(gw4, b_tab, w2d, b2s)

    return out[:n]


# speedup vs baseline: 1.3089x; 1.3089x over previous
import functools

import jax
import jax.numpy as jnp
from jax import lax
from jax.experimental import pallas as pl
from jax.experimental.pallas import tpu as pltpu
from jax.experimental.pallas import tpu_sc as plsc

_NC = 2
_NS = 16
_NW = _NC * _NS

_DEG = 16
_H = 32
_DF = 32
_PK = 4
_LW = _PK * _H

_CH = 128
_KG = 8


def _phase0_body(y_ref, w1a_ref, w1b4_ref, b14_ref, a_ref, b_ref):
    yb = y_ref[...]
    a_ref[...] = jnp.dot(yb, w1a_ref[...], preferred_element_type=jnp.float32)
    b_ref[...] = (
        jnp.dot(yb, w1b4_ref[...], preferred_element_type=jnp.float32) + b14_ref[...]
    )


def _phase2_body(ga_ref, gf_ref, b4_ref, w2d_ref, b2_ref, out_ref):
    nb = b4_ref.shape[0]
    b4 = b4_ref[...]
    w2d = w2d_ref[...]
    acc = jnp.zeros((nb, _LW), jnp.float32)
    for c in range(_DEG // _PK):
        a4 = ga_ref[pl.ds(c * nb, nb), :]
        f4 = gf_ref[pl.ds(c * nb, nb), :]
        h = jax.nn.gelu(a4 + b4)
        k = jnp.dot(h, w2d, preferred_element_type=jnp.float32) + b2_ref[...]
        acc = acc + k * f4
    s = (
        acc[:, 0:_H] + acc[:, _H : 2 * _H] + acc[:, 2 * _H : 3 * _H] + acc[:, 3 * _H :]
    )
    out_ref[...] = s * (1.0 / _DEG)


def _sc_gather2(ta, tf, idx_t, n_nodes, nb):
    n_edges = n_nodes * _DEG
    prows = n_edges // _PK
    prows_per_tile = prows // _NW
    prows_per_step = 256
    steps = prows_per_tile // prows_per_step
    cg = _DEG // _PK

    mesh = plsc.VectorSubcoreMesh(core_axis_name="c", subcore_axis_name="s")

    @functools.partial(
        pl.kernel,
        out_type=(
            jax.ShapeDtypeStruct((prows, _LW), jnp.float32),
            jax.ShapeDtypeStruct((prows, _LW), jnp.float32),
        ),
        mesh=mesh,
        scratch_types=[
            [pltpu.VMEM((prows_per_step,), jnp.int32) for _ in range(_PK)],
            pltpu.VMEM((_PK * prows_per_step, _H), jnp.float32),
            pltpu.VMEM((_PK * prows_per_step, _DF), jnp.float32),
            pltpu.SemaphoreType.DMA,
        ],
        compiler_params=pltpu.CompilerParams(use_tc_tiling_on_sc=False),
    )
    def gather_kernel(ta_hbm, tf_hbm, idx_hbm, ga_hbm, gf_hbm, idx_vs, ra_v, rf_v, sem):
        wid = lax.axis_index("s") * _NC + lax.axis_index("c")
        prow_base = wid * prows_per_tile

        def step(t, carry):
            poff = prow_base + t * prows_per_step
            seg = poff // nb
            m0 = poff % nb
            c = seg % cg
            node0 = (seg // cg) * nb + m0
            for q in range(_PK):
                pltpu.sync_copy(
                    idx_hbm.at[c * _PK + q, pl.ds(node0, prows_per_step)],
                    idx_vs[q],
                )
            cps = []
            for q in range(_PK):
                sl = pl.ds(q * prows_per_step, prows_per_step)
                cps.append(pltpu.async_copy(ta_hbm.at[idx_vs[q]], ra_v.at[sl], sem))
                cps.append(pltpu.async_copy(tf_hbm.at[idx_vs[q]], rf_v.at[sl], sem))
            for cp in cps:
                cp.wait()
            for q in range(_PK):
                sl = pl.ds(q * prows_per_step, prows_per_step)
                dst = (pl.ds(poff, prows_per_step), pl.ds(q * _H, _H))
                pltpu.sync_copy(ra_v.at[sl], ga_hbm.at[dst])
                pltpu.sync_copy(rf_v.at[sl], gf_hbm.at[dst])
            return carry

        lax.fori_loop(0, steps, step, None)

    return gather_kernel(ta, tf, idx_t)


def kernel(y, f_y, neighbors_index, neighbors_row_splits, W1, b1, W2, b2):
    del neighbors_row_splits
    n = y.shape[0]
    e = neighbors_index.shape[0]
    idx = neighbors_index.astype(jnp.int32)
    nb = 1024
    n_pad = ((n + nb - 1) // nb) * nb
    y_p = jnp.pad(y, ((0, n_pad - n), (0, 0)))
    idx_t = jnp.pad(idx.reshape(n, _DEG), ((0, n_pad - n), (0, 0))).T

    w1a = W1[:3]
    w1b4 = jnp.tile(W1[3:], (1, _PK))
    b14 = jnp.tile(b1, _PK).reshape(1, _LW)
    w2d = jnp.kron(jnp.eye(_PK, dtype=jnp.float32), W2)
    b2r = jnp.tile(b2, _PK).reshape(1, _LW)

    nb0 = nb
    a_tab, b_tab = pl.pallas_call(
        _phase0_body,
        grid=(n_pad // nb0,),
        in_specs=[
            pl.BlockSpec((nb0, 3), lambda i: (i, 0)),
            pl.BlockSpec((3, _H), lambda i: (0, 0)),
            pl.BlockSpec((3, _LW), lambda i: (0, 0)),
            pl.BlockSpec((1, _LW), lambda i: (0, 0)),
        ],
        out_specs=[
            pl.BlockSpec((nb0, _H), lambda i: (i, 0)),
            pl.BlockSpec((nb0, _LW), lambda i: (i, 0)),
        ],
        out_shape=[
            jax.ShapeDtypeStruct((n_pad, _H), jnp.float32),
            jax.ShapeDtypeStruct((n_pad, _LW), jnp.float32),
        ],
    )(y_p, w1a, w1b4, b14)

    ga4, gf4 = _sc_gather2(a_tab, f_y, idx_t, n_pad, nb)

    nr = nb * _DEG // _PK
    out = pl.pallas_call(
        _phase2_body,
        grid=(n_pad // nb,),
        in_specs=[
            pl.BlockSpec((nr, _LW), lambda i: (i, 0)),
            pl.BlockSpec((nr, _LW), lambda i: (i, 0)),
            pl.BlockSpec((nb, _LW), lambda i: (i, 0)),
            pl.BlockSpec((_LW, _LW), lambda i: (0, 0)),
            pl.BlockSpec((1, _LW), lambda i: (0, 0)),
        ],
        out_specs=pl.BlockSpec((nb, _DF), lambda i: (i, 0)),
        out_shape=jax.ShapeDtypeStruct((n_pad, _DF), jnp.float32),
    )(ga4, gf4, b_tab, w2d, b2r)

    return out[:n]
